# baseline (speedup 1.0000x reference)
import jax
import jax.numpy as jnp
from jax.experimental import pallas as pl
from jax.experimental.pallas import tpu as pltpu

SIZE = 32
V = SIZE * SIZE
C = 256
B = 64
NB = 8


def _fused_kernel(a_ref, x_ref, g_ref, lp_ref, idx_ref, lpv_ref, xc_ref):
    b = pl.program_id(0)
    rows = []
    for i in range(NB):
        Xi = x_ref[i]
        ai = a_ref[i, :].reshape(1, C)
        rows.append(jax.lax.dot_general(
            ai, Xi, (((1,), (1,)), ((), ()))))
    xc_ref[pl.ds(b * NB, NB), :] = jnp.concatenate(rows, axis=0)

    def _tail(blk):
        base = blk * NB
        xc = xc_ref[pl.ds(base, NB), :]
        m = jnp.max(xc, axis=1, keepdims=True)
        lse = jnp.log(jnp.sum(jnp.exp(xc - m), axis=1, keepdims=True)) + m
        lp = xc - lse
        lp_ref[pl.ds(base, NB), :] = lp
        s = lp + g_ref[pl.ds(base, NB), :]
        smax = jnp.max(s, axis=1, keepdims=True)
        iota = jax.lax.broadcasted_iota(jnp.int32, (NB, V), 1)
        idx = jnp.min(jnp.where(s == smax, iota, V), axis=1, keepdims=True)
        idx_ref[pl.ds(base, NB), :] = idx
        lpv_ref[pl.ds(base, NB), :] = jnp.sum(
            jnp.where(iota == idx, lp, 0.0), axis=1, keepdims=True)

    @pl.when(b > 0)
    def _pipelined_tail():
        _tail(b - 1)

    @pl.when(b == B // NB - 1)
    def _last_tail():
        _tail(b)


def kernel(x, embedded_a):
    xt = x.transpose(0, 2, 3, 1).reshape(B, V, C)
    g = jax.random.gumbel(jax.random.key(42), (B, V), dtype=jnp.float32)
    lp, idx, lpv = pl.pallas_call(
        _fused_kernel,
        grid=(B // NB,),
        in_specs=[
            pl.BlockSpec((NB, C), lambda b: (b, 0)),
            pl.BlockSpec((NB, V, C), lambda b: (b, 0, 0)),
            pl.BlockSpec((B, V), lambda b: (0, 0)),
        ],
        out_specs=[
            pl.BlockSpec((B, V), lambda b: (0, 0)),
            pl.BlockSpec((B, 1), lambda b: (0, 0)),
            pl.BlockSpec((B, 1), lambda b: (0, 0)),
        ],
        out_shape=[
            jax.ShapeDtypeStruct((B, V), jnp.float32),
            jax.ShapeDtypeStruct((B, 1), jnp.int32),
            jax.ShapeDtypeStruct((B, 1), jnp.float32),
        ],
        scratch_shapes=[pltpu.VMEM((B, V), jnp.float32)],
        compiler_params=pltpu.CompilerParams(
            dimension_semantics=("arbitrary",),
        ),
    )(embedded_a, xt, g)
    idx = idx[:, 0]
    arg_lst = jnp.stack([idx % SIZE, idx // SIZE], axis=-1)
    return (arg_lst, lpv[:, 0], lp)
